# Initial kernel scaffold; baseline (speedup 1.0000x reference)
import jax
import jax.numpy as jnp
from jax.experimental import pallas as pl


def kernel(S):
    raise NotImplementedError("write your pallas kernel here")


# speedup vs baseline: 58.3342x; 58.3342x over previous
import jax
import jax.numpy as jnp
from jax.experimental import pallas as pl
from jax.experimental.pallas import tpu as pltpu

_K = 17
_PAD = (_K - 1) // 2
_TT = 256


def _batcher_pairs(n):
    pairs = []

    def merge(lo, m, r):
        step = r * 2
        if step < m:
            merge(lo, m, step)
            merge(lo + r, m, step)
            for i in range(lo + r, lo + m - r, step):
                pairs.append((i, i + r))
        else:
            pairs.append((lo, lo + r))

    def sort(lo, m):
        if m > 1:
            h = m // 2
            sort(lo, h)
            sort(lo + h, h)
            merge(lo, m, 1)

    sort(0, n)
    return pairs


def _median17_network():
    n = 32
    state = [(True, False)] * _K + [(False, True)] * (n - _K)
    perm = list(range(n))
    ops = []
    for (i, j) in _batcher_pairs(n):
        ri, ci = state[i]
        rj, cj = state[j]
        if not rj:
            continue
        if not ri:
            perm[i], perm[j] = perm[j], perm[i]
            state[i], state[j] = state[j], state[i]
            continue
        ops.append(("ce", perm[i], perm[j]))
        state[i] = (ri or rj, ci and cj)
        state[j] = (ri or rj, ci or cj)
    out_slot = perm[_K // 2]
    needed = {out_slot}
    kept = []
    for op in reversed(ops):
        _, a, b = op
        if a in needed or b in needed:
            kept.append(op)
            needed.add(a)
            needed.add(b)
    kept.reverse()
    return kept, out_slot


_MEDIAN_OPS, _MEDIAN_OUT = _median17_network()


def _median17(vals):
    vals = list(vals)
    for _, a, b in _MEDIAN_OPS:
        va, vb = vals[a], vals[b]
        vals[a] = jnp.minimum(va, vb)
        vals[b] = jnp.maximum(va, vb)
    return vals[_MEDIAN_OUT]


def _hpss_kernel(x_ref, oh_ref, op_ref):
    t = pl.program_id(1)
    col0 = pl.multiple_of(t * _TT, 128)
    f = oh_ref.shape[1]

    big = x_ref[0, :, pl.ds(col0, _TT + 128)]
    rows = big[_PAD:_PAD + f, :]
    harm = _median17([rows[:, i:i + _TT] for i in range(_K)])
    mid = big[:, _PAD:_PAD + _TT]
    perc = _median17([mid[i:i + f, :] for i in range(_K)])
    s = mid[_PAD:_PAD + f, :]

    z = jnp.maximum(harm, perc)
    tiny = jnp.finfo(jnp.float32).tiny
    z = jnp.where(z < tiny, jnp.float32(1.0), z)
    qh = harm / z
    qp = perc / z
    m = qh * qh
    r = qp * qp
    denom = m + r
    oh_ref[0] = s * (m / denom)
    op_ref[0] = s * (r / denom)


def kernel(S):
    B, C, F, T = S.shape
    x = S.reshape(B * C, F, T)
    rpad = 128 - _PAD
    wp = T + 128
    xp = jnp.pad(x, ((0, 0), (_PAD, _PAD), (_PAD, rpad)))
    nt = T // _TT
    outs = pl.pallas_call(
        _hpss_kernel,
        grid=(B * C, nt),
        in_specs=[
            pl.BlockSpec((1, F + 2 * _PAD, wp), lambda b, t: (b, 0, 0))
        ],
        out_specs=[
            pl.BlockSpec((1, F, _TT), lambda b, t: (b, 0, t)),
            pl.BlockSpec((1, F, _TT), lambda b, t: (b, 0, t)),
        ],
        out_shape=[
            jax.ShapeDtypeStruct((B * C, F, T), S.dtype),
            jax.ShapeDtypeStruct((B * C, F, T), S.dtype),
        ],
        compiler_params=pltpu.CompilerParams(
            dimension_semantics=("parallel", "arbitrary"),
        ),
        name="hpss_fused",
    )(xp)
    oh, op_ = outs
    return oh.reshape(B, C, F, T), op_.reshape(B, C, F, T)
